# baseline (speedup 1.0000x reference)
import jax
import jax.numpy as jnp
from jax import lax
from jax.experimental import pallas as pl
from jax.experimental.pallas import tpu as pltpu
from jax.experimental.pallas import tpu_sc as plsc

try:
    _INFO = plsc.get_sparse_core_info()
    _NC = _INFO.num_cores
    _NS = _INFO.num_subcores
    _LANES = _INFO.num_lanes
except Exception:
    _NC, _NS, _LANES = 2, 16, 16
_NW = _NC * _NS


def _make_bias_body(b_per_w, groups):
    def _bias_body(users_hbm, items_hbm, bu_tbl, bi_tbl, mu_hbm,
                   bsum_hbm,
                   idx_u, idx_i, bu_v, bi_v, mu_v, out_v,
                   sem_bu, sem_bi):
        wid = lax.axis_index("s") * _NC + lax.axis_index("c")
        base = wid * b_per_w

        pltpu.sync_copy(users_hbm.at[pl.ds(base, b_per_w)], idx_u)
        pltpu.sync_copy(items_hbm.at[pl.ds(base, b_per_w)], idx_i)
        pltpu.sync_copy(mu_hbm, mu_v)

        cbu = pltpu.async_copy(bu_tbl.at[idx_u], bu_v, sem_bu)
        cbi = pltpu.async_copy(bi_tbl.at[idx_i], bi_v, sem_bi)
        cbu.wait()
        cbi.wait()

        mu_vec = mu_v[...]
        for g in range(groups):
            sl = pl.ds(g * _LANES, _LANES)
            out_v[sl] = bu_v[sl] + bi_v[sl] + mu_vec

        pltpu.sync_copy(out_v, bsum_hbm.at[pl.ds(base, b_per_w)])

    return _bias_body


def _make_main_body(b_per_w, factors, groups):
    n_chunks = 4
    b_chunk = b_per_w // n_chunks
    groups_c = groups // n_chunks

    def _main_body(users_hbm, items_hbm, u_tbl, m_tbl, bsum_hbm,
                   out_hbm,
                   idx_u, idx_i, u_rows, m_rows, bsum_v, out_v,
                   sem_u0, sem_u1, sem_m0, sem_m1):
        wid = lax.axis_index("s") * _NC + lax.axis_index("c")
        base = wid * b_per_w
        sems_u = (sem_u0, sem_u1)
        sems_m = (sem_m0, sem_m1)

        pltpu.sync_copy(users_hbm.at[pl.ds(base, b_per_w)], idx_u)
        pltpu.sync_copy(items_hbm.at[pl.ds(base, b_per_w)], idx_i)
        pltpu.sync_copy(bsum_hbm.at[pl.ds(base, b_per_w)], bsum_v)

        lane = lax.iota(jnp.int32, _LANES)

        def fire(c):
            cb = c * b_chunk
            slot = c % 2

            def fetch_chunk(g, carry):
                vu = idx_u[pl.ds(cb + g * _LANES, _LANES)]
                vm = idx_i[pl.ds(cb + g * _LANES, _LANES)]
                for j in range(_LANES):
                    e = g * _LANES + j
                    pltpu.async_copy(u_tbl.at[vu[j]],
                                     u_rows.at[slot].at[e], sems_u[slot])
                    pltpu.async_copy(m_tbl.at[vm[j]],
                                     m_rows.at[slot].at[e], sems_m[slot])
                return carry

            lax.fori_loop(0, groups_c, fetch_chunk, 0)

        def drain(c):
            slot = c % 2
            pltpu.make_async_copy(u_tbl.at[pl.ds(0, b_chunk)],
                                  u_rows.at[slot], sems_u[slot]).wait()
            pltpu.make_async_copy(m_tbl.at[pl.ds(0, b_chunk)],
                                  m_rows.at[slot], sems_m[slot]).wait()

        def compute(c):
            cb = c * b_chunk
            slot = c % 2

            def group_body(g, carry):
                rows = g * _LANES + lane
                sl = pl.ds(cb + g * _LANES, _LANES)
                acc = jnp.zeros((_LANES,), jnp.float32)
                for f in range(factors):
                    col = jnp.full((_LANES,), f, jnp.int32)
                    pu = plsc.load_gather(u_rows.at[slot], [rows, col])
                    pm = plsc.load_gather(m_rows.at[slot], [rows, col])
                    acc = acc + pu * pm
                out_v[sl] = acc + bsum_v[sl]
                return carry

            lax.fori_loop(0, groups_c, group_body, 0)

        fire(0)
        fire(1)
        drain(0)
        compute(0)
        fire(2)
        drain(1)
        compute(1)
        fire(3)
        drain(2)
        compute(2)
        drain(3)
        compute(3)

        pltpu.sync_copy(out_v, out_hbm.at[pl.ds(base, b_per_w)])

    return _main_body


def _build(batch, factors):
    b_per_w = batch // _NW
    groups = b_per_w // _LANES
    mesh = plsc.VectorSubcoreMesh(
        core_axis_name="c", subcore_axis_name="s",
        num_cores=_NC, num_subcores=_NS)
    bias_run = pl.kernel(
        _make_bias_body(b_per_w, groups),
        out_type=jax.ShapeDtypeStruct((batch,), jnp.float32),
        mesh=mesh,
        scratch_types=[
            pltpu.VMEM((b_per_w,), jnp.int32),
            pltpu.VMEM((b_per_w,), jnp.int32),
            pltpu.VMEM((b_per_w,), jnp.float32),
            pltpu.VMEM((b_per_w,), jnp.float32),
            pltpu.VMEM((_LANES,), jnp.float32),
            pltpu.VMEM((b_per_w,), jnp.float32),
            pltpu.SemaphoreType.DMA,
            pltpu.SemaphoreType.DMA,
        ],
        compiler_params=pltpu.CompilerParams(
            needs_layout_passes=False, use_tc_tiling_on_sc=False),
    )
    main_run = pl.kernel(
        _make_main_body(b_per_w, factors, groups),
        out_type=jax.ShapeDtypeStruct((batch,), jnp.float32),
        mesh=mesh,
        scratch_types=[
            pltpu.VMEM((b_per_w,), jnp.int32),
            pltpu.VMEM((b_per_w,), jnp.int32),
            pltpu.VMEM((2, b_per_w // 4, factors), jnp.float32),
            pltpu.VMEM((2, b_per_w // 4, factors), jnp.float32),
            pltpu.VMEM((b_per_w,), jnp.float32),
            pltpu.VMEM((b_per_w,), jnp.float32),
            pltpu.SemaphoreType.DMA,
            pltpu.SemaphoreType.DMA,
            pltpu.SemaphoreType.DMA,
            pltpu.SemaphoreType.DMA,
        ],
        compiler_params=pltpu.CompilerParams(needs_layout_passes=False),
    )
    return bias_run, main_run


_TP_BLK = 2048


def _tp_body(in_ref, out_ref):
    x = in_ref[...]
    f = x.shape[0]
    eye = (lax.broadcasted_iota(jnp.int32, (f, f), 0)
           == lax.broadcasted_iota(jnp.int32, (f, f), 1)).astype(x.dtype)
    out_ref[...] = lax.dot_general(
        x, eye, (((0,), (0,)), ((), ())),
        preferred_element_type=jnp.float32)


def _transpose(table_t):
    f, n = table_t.shape
    grid = (pl.cdiv(n, _TP_BLK),)
    return pl.pallas_call(
        _tp_body,
        grid=grid,
        in_specs=[pl.BlockSpec((f, _TP_BLK), lambda i: (0, i))],
        out_specs=pl.BlockSpec((_TP_BLK, f), lambda i: (i, 0)),
        out_shape=jax.ShapeDtypeStruct((n, f), table_t.dtype),
    )(table_t)


@jax.jit
def kernel(users, items, U, M, BU, BI, mu):
    users = users.astype(jnp.int32)
    items = items.astype(jnp.int32)
    bu_flat = BU.reshape((BU.shape[0],))
    bi_flat = BI.reshape((BI.shape[0],))
    mu16 = jnp.broadcast_to(mu, (_LANES,))
    u_rm = _transpose(U.T)
    m_rm = _transpose(M.T)
    bias_run, main_run = _build(users.shape[0], U.shape[1])
    bsum = bias_run(users, items, bu_flat, bi_flat, mu16)
    return main_run(users, items, u_rm, m_rm, bsum)


# speedup vs baseline: 1.6278x; 1.6278x over previous
import jax
import jax.numpy as jnp
from jax import lax
from jax.experimental import pallas as pl
from jax.experimental.pallas import tpu as pltpu
from jax.experimental.pallas import tpu_sc as plsc

try:
    _INFO = plsc.get_sparse_core_info()
    _NC = _INFO.num_cores
    _NS = _INFO.num_subcores
    _LANES = _INFO.num_lanes
except Exception:
    _NC, _NS, _LANES = 2, 16, 16
_NW = _NC * _NS


def _make_bias_body(b_per_w, groups):
    def _bias_body(users_hbm, items_hbm, bu_tbl, bi_tbl, mu_hbm,
                   bsum_hbm,
                   idx_u, idx_i, bu_v, bi_v, mu_v, out_v,
                   sem_bu, sem_bi):
        wid = lax.axis_index("s") * _NC + lax.axis_index("c")
        base = wid * b_per_w

        pltpu.sync_copy(users_hbm.at[pl.ds(base, b_per_w)], idx_u)
        pltpu.sync_copy(items_hbm.at[pl.ds(base, b_per_w)], idx_i)
        pltpu.sync_copy(mu_hbm, mu_v)

        cbu = pltpu.async_copy(bu_tbl.at[idx_u], bu_v, sem_bu)
        cbi = pltpu.async_copy(bi_tbl.at[idx_i], bi_v, sem_bi)
        cbu.wait()
        cbi.wait()

        mu_vec = mu_v[...]
        for g in range(groups):
            sl = pl.ds(g * _LANES, _LANES)
            out_v[sl] = bu_v[sl] + bi_v[sl] + mu_vec

        pltpu.sync_copy(out_v, bsum_hbm.at[pl.ds(base, b_per_w)])

    return _bias_body


def _make_main_body(b_per_w, factors, groups):
    n_chunks = 4
    b_chunk = b_per_w // n_chunks
    groups_c = groups // n_chunks

    def _main_body(users_hbm, items_hbm, u_tbl, m_tbl, bsum_hbm,
                   out_hbm,
                   idx_u, idx_i, u_rows, m_rows, bsum_v, out_v,
                   sem_u0, sem_u1, sem_m0, sem_m1):
        wid = lax.axis_index("s") * _NC + lax.axis_index("c")
        base = wid * b_per_w
        sems_u = (sem_u0, sem_u1)
        sems_m = (sem_m0, sem_m1)

        pltpu.sync_copy(users_hbm.at[pl.ds(base, b_per_w)], idx_u)
        pltpu.sync_copy(items_hbm.at[pl.ds(base, b_per_w)], idx_i)
        pltpu.sync_copy(bsum_hbm.at[pl.ds(base, b_per_w)], bsum_v)

        lane = lax.iota(jnp.int32, _LANES)

        def fire(c):
            cb = c * b_chunk
            slot = c % 2

            def fetch_chunk(g, carry):
                vu = idx_u[pl.ds(cb + g * _LANES, _LANES)]
                vm = idx_i[pl.ds(cb + g * _LANES, _LANES)]
                for j in range(_LANES):
                    e = g * _LANES + j
                    pltpu.async_copy(u_tbl.at[vu[j]],
                                     u_rows.at[slot].at[e], sems_u[slot])
                    pltpu.async_copy(m_tbl.at[vm[j]],
                                     m_rows.at[slot].at[e], sems_m[slot])
                return carry

            lax.fori_loop(0, groups_c, fetch_chunk, 0)

        def drain(c):
            slot = c % 2
            pltpu.make_async_copy(u_tbl.at[pl.ds(0, b_chunk)],
                                  u_rows.at[slot], sems_u[slot]).wait()
            pltpu.make_async_copy(m_tbl.at[pl.ds(0, b_chunk)],
                                  m_rows.at[slot], sems_m[slot]).wait()

        def compute(c):
            cb = c * b_chunk
            slot = c % 2

            def group_body(g, carry):
                rows = g * _LANES + lane
                sl = pl.ds(cb + g * _LANES, _LANES)
                acc = jnp.zeros((_LANES,), jnp.float32)
                for f in range(factors):
                    col = jnp.full((_LANES,), f, jnp.int32)
                    pu = plsc.load_gather(u_rows.at[slot], [rows, col])
                    pm = plsc.load_gather(m_rows.at[slot], [rows, col])
                    acc = acc + pu * pm
                out_v[sl] = acc + bsum_v[sl]
                return carry

            lax.fori_loop(0, groups_c, group_body, 0)

        fire(0)
        fire(1)
        drain(0)
        compute(0)
        fire(2)
        drain(1)
        compute(1)
        fire(3)
        drain(2)
        compute(2)
        drain(3)
        compute(3)

        pltpu.sync_copy(out_v, out_hbm.at[pl.ds(base, b_per_w)])

    return _main_body


def _build(batch, factors):
    b_per_w = batch // _NW
    groups = b_per_w // _LANES
    mesh = plsc.VectorSubcoreMesh(
        core_axis_name="c", subcore_axis_name="s",
        num_cores=_NC, num_subcores=_NS)
    bias_run = pl.kernel(
        _make_bias_body(b_per_w, groups),
        out_type=jax.ShapeDtypeStruct((batch,), jnp.float32),
        mesh=mesh,
        scratch_types=[
            pltpu.VMEM((b_per_w,), jnp.int32),
            pltpu.VMEM((b_per_w,), jnp.int32),
            pltpu.VMEM((b_per_w,), jnp.float32),
            pltpu.VMEM((b_per_w,), jnp.float32),
            pltpu.VMEM((_LANES,), jnp.float32),
            pltpu.VMEM((b_per_w,), jnp.float32),
            pltpu.SemaphoreType.DMA,
            pltpu.SemaphoreType.DMA,
        ],
        compiler_params=pltpu.CompilerParams(
            needs_layout_passes=False, use_tc_tiling_on_sc=False),
    )
    main_run = pl.kernel(
        _make_main_body(b_per_w, factors, groups),
        out_type=jax.ShapeDtypeStruct((batch,), jnp.float32),
        mesh=mesh,
        scratch_types=[
            pltpu.VMEM((b_per_w,), jnp.int32),
            pltpu.VMEM((b_per_w,), jnp.int32),
            pltpu.VMEM((2, b_per_w // 4, factors), jnp.float32),
            pltpu.VMEM((2, b_per_w // 4, factors), jnp.float32),
            pltpu.VMEM((b_per_w,), jnp.float32),
            pltpu.VMEM((b_per_w,), jnp.float32),
            pltpu.SemaphoreType.DMA,
            pltpu.SemaphoreType.DMA,
            pltpu.SemaphoreType.DMA,
            pltpu.SemaphoreType.DMA,
        ],
        compiler_params=pltpu.CompilerParams(needs_layout_passes=False),
    )
    return bias_run, main_run


_TP_BLK = 8192


def _tp_body(in_ref, out_ref):
    x = in_ref[...]
    f = x.shape[0]
    eye = (lax.broadcasted_iota(jnp.int32, (f, f), 0)
           == lax.broadcasted_iota(jnp.int32, (f, f), 1)).astype(x.dtype)
    out_ref[...] = lax.dot_general(
        x, eye, (((0,), (0,)), ((), ())),
        preferred_element_type=jnp.float32)


def _transpose(table_t):
    f, n = table_t.shape
    grid = (pl.cdiv(n, _TP_BLK),)
    return pl.pallas_call(
        _tp_body,
        grid=grid,
        in_specs=[pl.BlockSpec((f, _TP_BLK), lambda i: (0, i))],
        out_specs=pl.BlockSpec((_TP_BLK, f), lambda i: (i, 0)),
        out_shape=jax.ShapeDtypeStruct((n, f), table_t.dtype),
    )(table_t)


@jax.jit
def kernel(users, items, U, M, BU, BI, mu):
    users = users.astype(jnp.int32)
    items = items.astype(jnp.int32)
    bu_flat = BU.reshape((BU.shape[0],))
    bi_flat = BI.reshape((BI.shape[0],))
    mu16 = jnp.broadcast_to(mu, (_LANES,))
    u_rm = _transpose(U.T)
    m_rm = _transpose(M.T)
    bias_run, main_run = _build(users.shape[0], U.shape[1])
    bsum = bias_run(users, items, bu_flat, bi_flat, mu16)
    return main_run(users, items, u_rm, m_rm, bsum)


# speedup vs baseline: 1.7989x; 1.1051x over previous
import jax
import jax.numpy as jnp
from jax import lax
from jax.experimental import pallas as pl
from jax.experimental.pallas import tpu as pltpu
from jax.experimental.pallas import tpu_sc as plsc

try:
    _INFO = plsc.get_sparse_core_info()
    _NC = _INFO.num_cores
    _NS = _INFO.num_subcores
    _LANES = _INFO.num_lanes
except Exception:
    _NC, _NS, _LANES = 2, 16, 16
_NW = _NC * _NS


def _make_bias_body(b_per_w, groups):
    def _bias_body(users_hbm, items_hbm, bu_tbl, bi_tbl, mu_hbm,
                   bsum_hbm,
                   idx_u, idx_i, bu_v, bi_v, mu_v, out_v,
                   sem_bu, sem_bi):
        wid = lax.axis_index("s") * _NC + lax.axis_index("c")
        base = wid * b_per_w

        pltpu.sync_copy(users_hbm.at[pl.ds(base, b_per_w)], idx_u)
        pltpu.sync_copy(items_hbm.at[pl.ds(base, b_per_w)], idx_i)
        pltpu.sync_copy(mu_hbm, mu_v)

        cbu = pltpu.async_copy(bu_tbl.at[idx_u], bu_v, sem_bu)
        cbi = pltpu.async_copy(bi_tbl.at[idx_i], bi_v, sem_bi)
        cbu.wait()
        cbi.wait()

        mu_vec = mu_v[...]
        for g in range(groups):
            sl = pl.ds(g * _LANES, _LANES)
            out_v[sl] = bu_v[sl] + bi_v[sl] + mu_vec

        pltpu.sync_copy(out_v, bsum_hbm.at[pl.ds(base, b_per_w)])

    return _bias_body


def _make_main_body(b_per_w, factors, groups):
    n_chunks = 4
    b_chunk = b_per_w // n_chunks
    groups_c = groups // n_chunks

    def _main_body(users_hbm, items_hbm, u_tbl, m_tbl, bsum_hbm,
                   out_hbm,
                   idx_u, idx_i, u_rows, m_rows, bsum_v, out_v,
                   sem_u0, sem_u1, sem_m0, sem_m1):
        wid = lax.axis_index("s") * _NC + lax.axis_index("c")
        base = wid * b_per_w
        sems_u = (sem_u0, sem_u1)
        sems_m = (sem_m0, sem_m1)

        pltpu.sync_copy(users_hbm.at[pl.ds(base, b_per_w)], idx_u)
        pltpu.sync_copy(items_hbm.at[pl.ds(base, b_per_w)], idx_i)
        pltpu.sync_copy(bsum_hbm.at[pl.ds(base, b_per_w)], bsum_v)

        lane = lax.iota(jnp.int32, _LANES)

        def fire(c):
            cb = c * b_chunk
            slot = c % 2

            def fetch_chunk(g, carry):
                vu = idx_u[pl.ds(cb + g * _LANES, _LANES)]
                vm = idx_i[pl.ds(cb + g * _LANES, _LANES)]
                for j in range(_LANES):
                    e = g * _LANES + j
                    pltpu.async_copy(u_tbl.at[vu[j]],
                                     u_rows.at[slot].at[e], sems_u[slot])
                    pltpu.async_copy(m_tbl.at[vm[j]],
                                     m_rows.at[slot].at[e], sems_m[slot])
                return carry

            lax.fori_loop(0, groups_c, fetch_chunk, 0)

        def drain(c):
            slot = c % 2
            pltpu.make_async_copy(u_tbl.at[pl.ds(0, b_chunk)],
                                  u_rows.at[slot], sems_u[slot]).wait()
            pltpu.make_async_copy(m_tbl.at[pl.ds(0, b_chunk)],
                                  m_rows.at[slot], sems_m[slot]).wait()

        def compute(c):
            cb = c * b_chunk
            slot = c % 2

            def group_body(g, carry):
                rows = g * _LANES + lane
                sl = pl.ds(cb + g * _LANES, _LANES)
                acc = jnp.zeros((_LANES,), jnp.float32)
                for f in range(factors):
                    col = jnp.full((_LANES,), f, jnp.int32)
                    pu = plsc.load_gather(u_rows.at[slot], [rows, col])
                    pm = plsc.load_gather(m_rows.at[slot], [rows, col])
                    acc = acc + pu * pm
                out_v[sl] = acc + bsum_v[sl]
                return carry

            lax.fori_loop(0, groups_c, group_body, 0)

        fire(0)
        fire(1)
        drain(0)
        compute(0)
        fire(2)
        drain(1)
        compute(1)
        fire(3)
        drain(2)
        compute(2)
        drain(3)
        compute(3)

        pltpu.sync_copy(out_v, out_hbm.at[pl.ds(base, b_per_w)])

    return _main_body


def _build(batch, factors):
    b_per_w = batch // _NW
    groups = b_per_w // _LANES
    mesh = plsc.VectorSubcoreMesh(
        core_axis_name="c", subcore_axis_name="s",
        num_cores=_NC, num_subcores=_NS)
    bias_run = pl.kernel(
        _make_bias_body(b_per_w, groups),
        out_type=jax.ShapeDtypeStruct((batch,), jnp.float32),
        mesh=mesh,
        scratch_types=[
            pltpu.VMEM((b_per_w,), jnp.int32),
            pltpu.VMEM((b_per_w,), jnp.int32),
            pltpu.VMEM((b_per_w,), jnp.float32),
            pltpu.VMEM((b_per_w,), jnp.float32),
            pltpu.VMEM((_LANES,), jnp.float32),
            pltpu.VMEM((b_per_w,), jnp.float32),
            pltpu.SemaphoreType.DMA,
            pltpu.SemaphoreType.DMA,
        ],
        compiler_params=pltpu.CompilerParams(
            needs_layout_passes=False, use_tc_tiling_on_sc=False),
    )
    main_run = pl.kernel(
        _make_main_body(b_per_w, factors, groups),
        out_type=jax.ShapeDtypeStruct((batch,), jnp.float32),
        mesh=mesh,
        scratch_types=[
            pltpu.VMEM((b_per_w,), jnp.int32),
            pltpu.VMEM((b_per_w,), jnp.int32),
            pltpu.VMEM((2, b_per_w // 4, factors), jnp.float32),
            pltpu.VMEM((2, b_per_w // 4, factors), jnp.float32),
            pltpu.VMEM((b_per_w,), jnp.float32),
            pltpu.VMEM((b_per_w,), jnp.float32),
            pltpu.SemaphoreType.DMA,
            pltpu.SemaphoreType.DMA,
            pltpu.SemaphoreType.DMA,
            pltpu.SemaphoreType.DMA,
        ],
        compiler_params=pltpu.CompilerParams(needs_layout_passes=False),
    )
    return bias_run, main_run


_TP_BLK = 32768


def _tp_body(in_ref, out_ref):
    x = in_ref[...]
    f = x.shape[0]
    eye = (lax.broadcasted_iota(jnp.int32, (f, f), 0)
           == lax.broadcasted_iota(jnp.int32, (f, f), 1)).astype(x.dtype)
    out_ref[...] = lax.dot_general(
        x, eye, (((0,), (0,)), ((), ())),
        preferred_element_type=jnp.float32)


def _transpose(table_t):
    f, n = table_t.shape
    grid = (pl.cdiv(n, _TP_BLK),)
    return pl.pallas_call(
        _tp_body,
        grid=grid,
        in_specs=[pl.BlockSpec((f, _TP_BLK), lambda i: (0, i))],
        out_specs=pl.BlockSpec((_TP_BLK, f), lambda i: (i, 0)),
        out_shape=jax.ShapeDtypeStruct((n, f), table_t.dtype),
    )(table_t)


@jax.jit
def kernel(users, items, U, M, BU, BI, mu):
    users = users.astype(jnp.int32)
    items = items.astype(jnp.int32)
    bu_flat = BU.reshape((BU.shape[0],))
    bi_flat = BI.reshape((BI.shape[0],))
    mu16 = jnp.broadcast_to(mu, (_LANES,))
    u_rm = _transpose(U.T)
    m_rm = _transpose(M.T)
    bias_run, main_run = _build(users.shape[0], U.shape[1])
    bsum = bias_run(users, items, bu_flat, bi_flat, mu16)
    return main_run(users, items, u_rm, m_rm, bsum)


# speedup vs baseline: 1.8275x; 1.0159x over previous
import jax
import jax.numpy as jnp
from jax import lax
from jax.experimental import pallas as pl
from jax.experimental.pallas import tpu as pltpu
from jax.experimental.pallas import tpu_sc as plsc

try:
    _INFO = plsc.get_sparse_core_info()
    _NC = _INFO.num_cores
    _NS = _INFO.num_subcores
    _LANES = _INFO.num_lanes
except Exception:
    _NC, _NS, _LANES = 2, 16, 16
_NW = _NC * _NS


def _make_bias_body(b_per_w, groups):
    def _bias_body(users_hbm, items_hbm, bu_t, bi_t, mu_hbm,
                   bsum_hbm,
                   idx_u, idx_i, bu_v, bi_v, mu_v, out_v,
                   sem_bu, sem_bi):
        bu_tbl = bu_t.at[0]
        bi_tbl = bi_t.at[0]
        wid = lax.axis_index("s") * _NC + lax.axis_index("c")
        base = wid * b_per_w

        pltpu.sync_copy(users_hbm.at[pl.ds(base, b_per_w)], idx_u)
        pltpu.sync_copy(items_hbm.at[pl.ds(base, b_per_w)], idx_i)
        pltpu.sync_copy(mu_hbm, mu_v)

        cbu = pltpu.async_copy(bu_tbl.at[idx_u], bu_v, sem_bu)
        cbi = pltpu.async_copy(bi_tbl.at[idx_i], bi_v, sem_bi)
        cbu.wait()
        cbi.wait()

        mu_vec = mu_v[...]
        for g in range(groups):
            sl = pl.ds(g * _LANES, _LANES)
            out_v[sl] = bu_v[sl] + bi_v[sl] + mu_vec

        pltpu.sync_copy(out_v, bsum_hbm.at[pl.ds(base, b_per_w)])

    return _bias_body


def _make_main_body(b_per_w, factors, groups):
    n_chunks = 4
    b_chunk = b_per_w // n_chunks
    groups_c = groups // n_chunks

    def _main_body(users_hbm, items_hbm, u_tbl, m_tbl, bsum_hbm,
                   out_hbm,
                   idx_u, idx_i, u_rows, m_rows, bsum_v, out_v,
                   sem_u0, sem_u1, sem_m0, sem_m1):
        wid = lax.axis_index("s") * _NC + lax.axis_index("c")
        base = wid * b_per_w
        sems_u = (sem_u0, sem_u1)
        sems_m = (sem_m0, sem_m1)

        pltpu.sync_copy(users_hbm.at[pl.ds(base, b_per_w)], idx_u)
        pltpu.sync_copy(items_hbm.at[pl.ds(base, b_per_w)], idx_i)
        pltpu.sync_copy(bsum_hbm.at[pl.ds(base, b_per_w)], bsum_v)

        lane = lax.iota(jnp.int32, _LANES)

        def fire(c):
            cb = c * b_chunk
            slot = c % 2

            def fetch_chunk(g, carry):
                vu = idx_u[pl.ds(cb + g * _LANES, _LANES)]
                vm = idx_i[pl.ds(cb + g * _LANES, _LANES)]
                for j in range(_LANES):
                    e = g * _LANES + j
                    pltpu.async_copy(u_tbl.at[vu[j]],
                                     u_rows.at[slot].at[e], sems_u[slot])
                    pltpu.async_copy(m_tbl.at[vm[j]],
                                     m_rows.at[slot].at[e], sems_m[slot])
                return carry

            lax.fori_loop(0, groups_c, fetch_chunk, 0)

        def drain(c):
            slot = c % 2
            pltpu.make_async_copy(u_tbl.at[pl.ds(0, b_chunk)],
                                  u_rows.at[slot], sems_u[slot]).wait()
            pltpu.make_async_copy(m_tbl.at[pl.ds(0, b_chunk)],
                                  m_rows.at[slot], sems_m[slot]).wait()

        def compute(c):
            cb = c * b_chunk
            slot = c % 2

            def group_body(g, carry):
                rows = g * _LANES + lane
                sl = pl.ds(cb + g * _LANES, _LANES)
                acc = jnp.zeros((_LANES,), jnp.float32)
                for f in range(factors):
                    col = jnp.full((_LANES,), f, jnp.int32)
                    pu = plsc.load_gather(u_rows.at[slot], [rows, col])
                    pm = plsc.load_gather(m_rows.at[slot], [rows, col])
                    acc = acc + pu * pm
                out_v[sl] = acc + bsum_v[sl]
                return carry

            lax.fori_loop(0, groups_c, group_body, 0)

        fire(0)
        fire(1)
        drain(0)
        compute(0)
        fire(2)
        drain(1)
        compute(1)
        fire(3)
        drain(2)
        compute(2)
        drain(3)
        compute(3)

        pltpu.sync_copy(out_v, out_hbm.at[pl.ds(base, b_per_w)])

    return _main_body


def _build(batch, factors):
    b_per_w = batch // _NW
    groups = b_per_w // _LANES
    mesh = plsc.VectorSubcoreMesh(
        core_axis_name="c", subcore_axis_name="s",
        num_cores=_NC, num_subcores=_NS)
    bias_run = pl.kernel(
        _make_bias_body(b_per_w, groups),
        out_type=jax.ShapeDtypeStruct((batch,), jnp.float32),
        mesh=mesh,
        scratch_types=[
            pltpu.VMEM((b_per_w,), jnp.int32),
            pltpu.VMEM((b_per_w,), jnp.int32),
            pltpu.VMEM((b_per_w,), jnp.float32),
            pltpu.VMEM((b_per_w,), jnp.float32),
            pltpu.VMEM((_LANES,), jnp.float32),
            pltpu.VMEM((b_per_w,), jnp.float32),
            pltpu.SemaphoreType.DMA,
            pltpu.SemaphoreType.DMA,
        ],
        compiler_params=pltpu.CompilerParams(
            needs_layout_passes=False, use_tc_tiling_on_sc=False),
    )
    main_run = pl.kernel(
        _make_main_body(b_per_w, factors, groups),
        out_type=jax.ShapeDtypeStruct((batch,), jnp.float32),
        mesh=mesh,
        scratch_types=[
            pltpu.VMEM((b_per_w,), jnp.int32),
            pltpu.VMEM((b_per_w,), jnp.int32),
            pltpu.VMEM((2, b_per_w // 4, factors), jnp.float32),
            pltpu.VMEM((2, b_per_w // 4, factors), jnp.float32),
            pltpu.VMEM((b_per_w,), jnp.float32),
            pltpu.VMEM((b_per_w,), jnp.float32),
            pltpu.SemaphoreType.DMA,
            pltpu.SemaphoreType.DMA,
            pltpu.SemaphoreType.DMA,
            pltpu.SemaphoreType.DMA,
        ],
        compiler_params=pltpu.CompilerParams(needs_layout_passes=False),
    )
    return bias_run, main_run


_TP_BLK = 32768


def _tp_body(in_ref, out_ref):
    x = in_ref[...]
    f = x.shape[0]
    eye = (lax.broadcasted_iota(jnp.int32, (f, f), 0)
           == lax.broadcasted_iota(jnp.int32, (f, f), 1)).astype(x.dtype)
    out_ref[...] = lax.dot_general(
        x, eye, (((0,), (0,)), ((), ())),
        preferred_element_type=jnp.float32)


def _transpose(table_t):
    f, n = table_t.shape
    grid = (pl.cdiv(n, _TP_BLK),)
    return pl.pallas_call(
        _tp_body,
        grid=grid,
        in_specs=[pl.BlockSpec((f, _TP_BLK), lambda i: (0, i))],
        out_specs=pl.BlockSpec((_TP_BLK, f), lambda i: (i, 0)),
        out_shape=jax.ShapeDtypeStruct((n, f), table_t.dtype),
    )(table_t)


@jax.jit
def kernel(users, items, U, M, BU, BI, mu):
    users = users.astype(jnp.int32)
    items = items.astype(jnp.int32)
    bu_flat = BU.T
    bi_flat = BI.T
    mu16 = jnp.broadcast_to(mu, (_LANES,))
    u_rm = _transpose(U.T)
    m_rm = _transpose(M.T)
    bias_run, main_run = _build(users.shape[0], U.shape[1])
    bsum = bias_run(users, items, bu_flat, bi_flat, mu16)
    return main_run(users, items, u_rm, m_rm, bsum)
